# baseline (speedup 1.0000x reference)
import functools

import jax
import jax.numpy as jnp
from jax import lax
from jax.experimental import pallas as pl
from jax.experimental.pallas import tpu as pltpu
from jax.experimental.pallas import tpu_sc as plsc

NUM_ENT = 50000
NUM_REL = 474
D = 128
NODE_D = 32
B = 128
NB = 10
THRESH = 1373

VOCAB_CHUNK = 2048

_G = 16
_NG = B // _G
_JSETS = ((0, 1, 2), (3, 4), (5, 6), (7, 8, 9))


def _nbr_fetch(src, t_flat):
    mesh = plsc.VectorSubcoreMesh(core_axis_name="c", subcore_axis_name="s",
                                  num_cores=2, num_subcores=16)

    @functools.partial(
        pl.kernel,
        out_type=jax.ShapeDtypeStruct((NB, B), jnp.int32),
        mesh=mesh,
        compiler_params=pltpu.CompilerParams(use_tc_tiling_on_sc=False),
        scratch_types=[
            pltpu.VMEM((_G,), jnp.int32),
            pltpu.VMEM((NB, _G), jnp.int32),
            pltpu.SemaphoreType.DMA,
            pltpu.SemaphoreType.DMA,
        ],
    )
    def k(src_h, tflat_h, nbr_out, src_v, out_v, sem, sem2):
        wid = lax.axis_index("s") * 2 + lax.axis_index("c")

        @pl.when(wid < _NG)
        def _():
            base = wid * _G
            pltpu.sync_copy(src_h.at[pl.ds(base, _G)], src_v)
            s = src_v[...]
            cps = [pltpu.async_copy(tflat_h.at[s + j * NUM_ENT],
                                    out_v.at[j], sem)
                   for j in range(NB)]
            for c in cps:
                c.wait()
            ocs = [pltpu.async_copy(out_v.at[j],
                                    nbr_out.at[j, pl.ds(base, _G)], sem2)
                   for j in range(NB)]
            for c in ocs:
                c.wait()

    return k(src, t_flat)


def _gather_stage(src, rel, nbrT, ent_embed, rel_embed):
    mesh = plsc.VectorSubcoreMesh(core_axis_name="c", subcore_axis_name="s",
                                  num_cores=2, num_subcores=16)
    f32 = jnp.float32

    @functools.partial(
        pl.kernel,
        out_type=[
            jax.ShapeDtypeStruct((B, D), f32),
            jax.ShapeDtypeStruct((B, D), f32),
            jax.ShapeDtypeStruct((B, D), f32),
            jax.ShapeDtypeStruct((B, D), f32),
            jax.ShapeDtypeStruct((B, D), f32),
            jax.ShapeDtypeStruct((B, D), f32),
            jax.ShapeDtypeStruct((B,), f32),
            jax.ShapeDtypeStruct((B,), f32),
            jax.ShapeDtypeStruct((B,), f32),
            jax.ShapeDtypeStruct((B,), f32),
        ],
        mesh=mesh,
        scratch_types=[
            pltpu.VMEM((_G,), jnp.int32),
            pltpu.VMEM((_G, D), f32),
            pltpu.VMEM((_G,), jnp.int32),
            pltpu.VMEM((_G,), jnp.int32),
            pltpu.VMEM((_G,), jnp.int32),
            pltpu.VMEM((3, _G, D), f32),
            pltpu.VMEM((_G, D), f32),
            pltpu.VMEM((_G,), f32),
            pltpu.SemaphoreType.DMA,
            pltpu.SemaphoreType.DMA,
        ],
    )
    def k(src_h, rel_h, tT_h, ent_h, relemb_h,
          h_out, r_out, es0_out, es1_out, es2_out, es3_out,
          c0_out, c1_out, c2_out, c3_out,
          id_v, hr_v, ix0, ix1, ix2, g_v, es_v, cnt_v, sem, sem2):
        wid = lax.axis_index("s") * 2 + lax.axis_index("c")
        grp = wid // 4
        role = wid % 4
        base = grp * _G
        ixs = (ix0, ix1, ix2)
        es_outs = (es0_out, es1_out, es2_out, es3_out)
        cnt_outs = (c0_out, c1_out, c2_out, c3_out)
        id_hs = (None, src_h, rel_h, None)
        emb_hs = (None, ent_h, relemb_h, None)
        row_outs = (None, h_out, r_out, None)

        for rr in range(4):
            @pl.when(role == rr)
            def _(rr=rr):
                jset = _JSETS[rr]
                icps = [pltpu.async_copy(tT_h.at[j, pl.ds(base, _G)],
                                         ixs[kk], sem2)
                        for kk, j in enumerate(jset)]
                if id_hs[rr] is not None:
                    icps.append(pltpu.async_copy(
                        id_hs[rr].at[pl.ds(base, _G)], id_v, sem2))
                for c in icps:
                    c.wait()
                cps = [pltpu.async_copy(ent_h.at[ixs[kk]], g_v.at[kk], sem)
                       for kk in range(len(jset))]
                if id_hs[rr] is not None:
                    cps.append(pltpu.async_copy(
                        emb_hs[rr].at[id_v], hr_v, sem))
                cnt = jnp.where(ix0[...] >= THRESH, 1.0, 0.0)
                for kk in range(1, len(jset)):
                    cnt = cnt + jnp.where(ixs[kk][...] >= THRESH, 1.0, 0.0)
                cnt_v[...] = cnt
                for c in cps:
                    c.wait()

                nj = len(jset)

                def acc_body(i, c):
                    for c8 in range(D // 16):
                        sl = pl.ds(c8 * 16, 16)
                        a = g_v[0, i, sl]
                        for kk in range(1, nj):
                            a = a + g_v[kk, i, sl]
                        es_v[i, sl] = a
                    return c
                lax.fori_loop(0, _G, acc_body, 0)

                ocs = [pltpu.async_copy(es_v, es_outs[rr].at[pl.ds(base, _G)],
                                        sem2),
                       pltpu.async_copy(cnt_v, cnt_outs[rr].at[pl.ds(base, _G)],
                                        sem2)]
                if id_hs[rr] is not None:
                    ocs.append(pltpu.async_copy(
                        hr_v, row_outs[rr].at[pl.ds(base, _G)], sem2))
                for c in ocs:
                    c.wait()

    return k(src, rel, nbrT, ent_embed, rel_embed)


def _dense_body(h_ref, r_ref, e0_ref, e1_ref, e2_ref, e3_ref,
                c0_ref, c1_ref, c2_ref, c3_ref, nod_ref, wve_ref, wvn_ref,
                bv_ref, f1a_ref, f1b_ref, b1_ref, f2_ref, b2_ref, ent_ref,
                yct_ref, out_s):
    @pl.when(pl.program_id(0) == 0)
    def _():
        e_sum = e0_ref[...] + e1_ref[...] + e2_ref[...] + e3_ref[...]
        cnt = c0_ref[...] + c1_ref[...] + c2_ref[...] + c3_ref[...]
        node = (NB - cnt) * nod_ref[0:1, :] + cnt * nod_ref[1:2, :]
        V = (h_ref[...]
             + jnp.dot(e_sum, wve_ref[...], preferred_element_type=jnp.float32)
             + jnp.dot(node, wvn_ref[...], preferred_element_type=jnp.float32)
             + NB * bv_ref[...])
        z1 = jnp.maximum(
            jnp.dot(V, f1a_ref[...], preferred_element_type=jnp.float32)
            + jnp.dot(r_ref[...], f1b_ref[...], preferred_element_type=jnp.float32)
            + b1_ref[...], 0.0)
        out_s[...] = (jnp.dot(z1, f2_ref[...], preferred_element_type=jnp.float32)
                      + b2_ref[...])

    logits = jax.lax.dot_general(ent_ref[...], out_s[...],
                                 (((1,), (1,)), ((), ())),
                                 precision=jax.lax.Precision.HIGHEST,
                                 preferred_element_type=jnp.float32)
    yct_ref[...] = jax.nn.sigmoid(logits)


def _dense_stage(h_emb, r_emb, es_parts, cnt_parts, nod_embed, WV, bV,
                 fc1_w, fc1_b, fc2_w, fc2_b, ent_embed):
    n_chunks = pl.cdiv(NUM_ENT, VOCAB_CHUNK)
    const = lambda shape: pl.BlockSpec(shape, lambda i: (0, 0))
    return pl.pallas_call(
        _dense_body,
        grid=(n_chunks,),
        in_specs=[
            const((B, D)),
            const((B, D)),
            const((B, D)), const((B, D)), const((B, D)), const((B, D)),
            const((B, 1)), const((B, 1)), const((B, 1)), const((B, 1)),
            const((2, NODE_D)),
            const((D, D)),
            const((NODE_D, D)),
            const((1, D)),
            const((D, D)),
            const((D, D)),
            const((1, D)),
            const((D, D)),
            const((1, D)),
            pl.BlockSpec((VOCAB_CHUNK, D), lambda i: (i, 0)),
        ],
        out_specs=pl.BlockSpec((VOCAB_CHUNK, B), lambda i: (i, 0)),
        out_shape=jax.ShapeDtypeStruct((NUM_ENT, B), jnp.float32),
        scratch_shapes=[pltpu.VMEM((B, D), jnp.float32)],
    )(h_emb, r_emb, *es_parts, *[c.reshape(B, 1) for c in cnt_parts],
      nod_embed,
      WV[:, :D].T, WV[:, D:].T, bV.reshape(1, D),
      fc1_w[:, :D].T, fc1_w[:, D:].T, fc1_b.reshape(1, D),
      fc2_w.T, fc2_b.reshape(1, D), ent_embed)


def kernel(src, rel, t_idxs, ent_embed, rel_embed, nod_embed,
           WQ, bQ, WK, bK, WV, bV, fc1_w, fc1_b, fc2_w, fc2_b):
    nbrT = _nbr_fetch(src, t_idxs.T.reshape(-1))
    outs = _gather_stage(src, rel, nbrT, ent_embed, rel_embed)
    h_emb, r_emb = outs[0], outs[1]
    es_parts, cnt_parts = outs[2:6], outs[6:10]
    yct = _dense_stage(h_emb, r_emb, es_parts, cnt_parts, nod_embed,
                       WV, bV, fc1_w, fc1_b, fc2_w, fc2_b, ent_embed)
    return yct.T


# speedup vs baseline: 1.1932x; 1.1932x over previous
import functools

import jax
import jax.numpy as jnp
from jax import lax
from jax.experimental import pallas as pl
from jax.experimental.pallas import tpu as pltpu
from jax.experimental.pallas import tpu_sc as plsc

NUM_ENT = 50000
NUM_REL = 474
D = 128
NODE_D = 32
B = 128
NB = 10
THRESH = 1373

VOCAB_CHUNK = 2048

_G = 16
_NG = B // _G
_JSETS = ((0, 1, 2), (3, 4), (5, 6), (7, 8, 9))


def _nbr_fetch(src, t_flat):
    mesh = plsc.VectorSubcoreMesh(core_axis_name="c", subcore_axis_name="s",
                                  num_cores=2, num_subcores=16)

    @functools.partial(
        pl.kernel,
        out_type=jax.ShapeDtypeStruct((NB, B), jnp.int32),
        mesh=mesh,
        compiler_params=pltpu.CompilerParams(use_tc_tiling_on_sc=False),
        scratch_types=[
            pltpu.VMEM((_G,), jnp.int32),
            pltpu.VMEM((NB, _G), jnp.int32),
            pltpu.SemaphoreType.DMA,
            pltpu.SemaphoreType.DMA,
        ],
    )
    def k(src_h, tflat_h, nbr_out, src_v, out_v, sem, sem2):
        wid = lax.axis_index("s") * 2 + lax.axis_index("c")

        @pl.when(wid < _NG)
        def _():
            base = wid * _G
            pltpu.sync_copy(src_h.at[pl.ds(base, _G)], src_v)
            s = src_v[...]
            cps = [pltpu.async_copy(tflat_h.at[s + j * NUM_ENT],
                                    out_v.at[j], sem)
                   for j in range(NB)]
            for c in cps:
                c.wait()
            ocs = [pltpu.async_copy(out_v.at[j],
                                    nbr_out.at[j, pl.ds(base, _G)], sem2)
                   for j in range(NB)]
            for c in ocs:
                c.wait()

    return k(src, t_flat)


def _gather_stage(src, rel, nbrT, ent_embed, rel_embed):
    mesh = plsc.VectorSubcoreMesh(core_axis_name="c", subcore_axis_name="s",
                                  num_cores=2, num_subcores=16)
    f32 = jnp.float32

    @functools.partial(
        pl.kernel,
        out_type=[
            jax.ShapeDtypeStruct((B, D), f32),
            jax.ShapeDtypeStruct((B, D), f32),
            jax.ShapeDtypeStruct((B, D), f32),
            jax.ShapeDtypeStruct((B, D), f32),
            jax.ShapeDtypeStruct((B, D), f32),
            jax.ShapeDtypeStruct((B, D), f32),
            jax.ShapeDtypeStruct((4, B), f32),
        ],
        mesh=mesh,
        scratch_types=[
            pltpu.VMEM((_G,), jnp.int32),
            pltpu.VMEM((_G, D), f32),
            pltpu.VMEM((_G,), jnp.int32),
            pltpu.VMEM((_G,), jnp.int32),
            pltpu.VMEM((_G,), jnp.int32),
            pltpu.VMEM((3, _G, D), f32),
            pltpu.VMEM((_G, D), f32),
            pltpu.VMEM((_G,), f32),
            pltpu.SemaphoreType.DMA,
            pltpu.SemaphoreType.DMA,
        ],
    )
    def k(src_h, rel_h, tT_h, ent_h, relemb_h,
          h_out, r_out, es0_out, es1_out, es2_out, es3_out, cnt4_out,
          id_v, hr_v, ix0, ix1, ix2, g_v, es_v, cnt_v, sem, sem2):
        wid = lax.axis_index("s") * 2 + lax.axis_index("c")
        grp = wid // 4
        role = wid % 4
        base = grp * _G
        ixs = (ix0, ix1, ix2)
        es_outs = (es0_out, es1_out, es2_out, es3_out)
        id_hs = (None, src_h, rel_h, None)
        emb_hs = (None, ent_h, relemb_h, None)
        row_outs = (None, h_out, r_out, None)

        for rr in range(4):
            @pl.when(role == rr)
            def _(rr=rr):
                jset = _JSETS[rr]
                icps = [pltpu.async_copy(tT_h.at[j, pl.ds(base, _G)],
                                         ixs[kk], sem2)
                        for kk, j in enumerate(jset)]
                if id_hs[rr] is not None:
                    icps.append(pltpu.async_copy(
                        id_hs[rr].at[pl.ds(base, _G)], id_v, sem2))
                for c in icps:
                    c.wait()
                cps = [pltpu.async_copy(ent_h.at[ixs[kk]], g_v.at[kk], sem)
                       for kk in range(len(jset))]
                if id_hs[rr] is not None:
                    cps.append(pltpu.async_copy(
                        emb_hs[rr].at[id_v], hr_v, sem))
                cnt = jnp.where(ix0[...] >= THRESH, 1.0, 0.0)
                for kk in range(1, len(jset)):
                    cnt = cnt + jnp.where(ixs[kk][...] >= THRESH, 1.0, 0.0)
                cnt_v[...] = cnt
                for c in cps:
                    c.wait()

                nj = len(jset)

                def acc_body(i, c):
                    for c8 in range(D // 16):
                        sl = pl.ds(c8 * 16, 16)
                        a = g_v[0, i, sl]
                        for kk in range(1, nj):
                            a = a + g_v[kk, i, sl]
                        es_v[i, sl] = a
                    return c
                lax.fori_loop(0, _G, acc_body, 0)

                ocs = [pltpu.async_copy(es_v, es_outs[rr].at[pl.ds(base, _G)],
                                        sem2),
                       pltpu.async_copy(
                           cnt_v, cnt4_out.at[rr, pl.ds(base, _G)], sem2)]
                if id_hs[rr] is not None:
                    ocs.append(pltpu.async_copy(
                        hr_v, row_outs[rr].at[pl.ds(base, _G)], sem2))
                for c in ocs:
                    c.wait()

    return k(src, rel, nbrT, ent_embed, rel_embed)


def _dense_body(h_ref, r_ref, e0_ref, e1_ref, e2_ref, e3_ref,
                cnt4_ref, w_ref, bias_ref, wve_ref,
                f1a_ref, f1b_ref, b1_ref, f2_ref, b2_ref, ent_ref,
                yct_ref, out_s):
    @pl.when(pl.program_id(0) == 0)
    def _():
        e_sum = e0_ref[...] + e1_ref[...] + e2_ref[...] + e3_ref[...]
        cnt_row = jnp.sum(cnt4_ref[...], axis=0, keepdims=True)
        node_v = jax.lax.dot_general(cnt_row, w_ref[...],
                                     (((0,), (0,)), ((), ())),
                                     preferred_element_type=jnp.float32)
        V = (h_ref[...]
             + jnp.dot(e_sum, wve_ref[...], preferred_element_type=jnp.float32)
             + node_v + bias_ref[...])
        z1 = jnp.maximum(
            jnp.dot(V, f1a_ref[...], preferred_element_type=jnp.float32)
            + jnp.dot(r_ref[...], f1b_ref[...], preferred_element_type=jnp.float32)
            + b1_ref[...], 0.0)
        out_s[...] = (jnp.dot(z1, f2_ref[...], preferred_element_type=jnp.float32)
                      + b2_ref[...])

    logits = jax.lax.dot_general(ent_ref[...], out_s[...],
                                 (((1,), (1,)), ((), ())),
                                 preferred_element_type=jnp.float32)
    yct_ref[...] = jax.nn.sigmoid(logits)


def _dense_stage(h_emb, r_emb, es_parts, cnt4, nod_embed, WV, bV,
                 fc1_w, fc1_b, fc2_w, fc2_b, ent_embed):
    WVnT = WV[:, D:].T
    w_row = ((nod_embed[1] - nod_embed[0]) @ WVnT).reshape(1, D)
    bias_row = (NB * (nod_embed[0] @ WVnT + bV)).reshape(1, D)
    n_chunks = pl.cdiv(NUM_ENT, VOCAB_CHUNK)
    const = lambda shape: pl.BlockSpec(shape, lambda i: (0, 0))
    return pl.pallas_call(
        _dense_body,
        grid=(n_chunks,),
        in_specs=[
            const((B, D)),
            const((B, D)),
            const((B, D)), const((B, D)), const((B, D)), const((B, D)),
            const((4, B)),
            const((1, D)),
            const((1, D)),
            const((D, D)),
            const((D, D)),
            const((D, D)),
            const((1, D)),
            const((D, D)),
            const((1, D)),
            pl.BlockSpec((VOCAB_CHUNK, D), lambda i: (i, 0)),
        ],
        out_specs=pl.BlockSpec((VOCAB_CHUNK, B), lambda i: (i, 0)),
        out_shape=jax.ShapeDtypeStruct((NUM_ENT, B), jnp.float32),
        scratch_shapes=[pltpu.VMEM((B, D), jnp.float32)],
    )(h_emb, r_emb, *es_parts, cnt4, w_row, bias_row,
      WV[:, :D].T,
      fc1_w[:, :D].T, fc1_w[:, D:].T, fc1_b.reshape(1, D),
      fc2_w.T, fc2_b.reshape(1, D), ent_embed)


def kernel(src, rel, t_idxs, ent_embed, rel_embed, nod_embed,
           WQ, bQ, WK, bK, WV, bV, fc1_w, fc1_b, fc2_w, fc2_b):
    nbrT = _nbr_fetch(src, t_idxs.T.reshape(-1))
    outs = _gather_stage(src, rel, nbrT, ent_embed, rel_embed)
    h_emb, r_emb = outs[0], outs[1]
    es_parts, cnt4 = outs[2:6], outs[6]
    yct = _dense_stage(h_emb, r_emb, es_parts, cnt4, nod_embed,
                       WV, bV, fc1_w, fc1_b, fc2_w, fc2_b, ent_embed)
    return yct.T


# speedup vs baseline: 1.4020x; 1.1750x over previous
import functools

import jax
import jax.numpy as jnp
from jax import lax
from jax.experimental import pallas as pl
from jax.experimental.pallas import tpu as pltpu
from jax.experimental.pallas import tpu_sc as plsc

NUM_ENT = 50000
NUM_REL = 474
D = 128
NODE_D = 32
B = 128
NB = 10
THRESH = 1373

VOCAB_CHUNK = 4096

_G = 16
_NG = B // _G
_JSETS = ((0, 1, 2), (3, 4), (5, 6), (7, 8, 9))


def _nbr_fetch(src, t_flat):
    mesh = plsc.VectorSubcoreMesh(core_axis_name="c", subcore_axis_name="s",
                                  num_cores=2, num_subcores=16)

    @functools.partial(
        pl.kernel,
        out_type=jax.ShapeDtypeStruct((NB, B), jnp.int32),
        mesh=mesh,
        compiler_params=pltpu.CompilerParams(use_tc_tiling_on_sc=False),
        scratch_types=[
            pltpu.VMEM((_G,), jnp.int32),
            pltpu.VMEM((NB, _G), jnp.int32),
            pltpu.SemaphoreType.DMA,
            pltpu.SemaphoreType.DMA,
        ],
    )
    def k(src_h, tflat_h, nbr_out, src_v, out_v, sem, sem2):
        wid = lax.axis_index("s") * 2 + lax.axis_index("c")

        @pl.when(wid < _NG)
        def _():
            base = wid * _G
            pltpu.sync_copy(src_h.at[pl.ds(base, _G)], src_v)
            s = src_v[...]
            cps = [pltpu.async_copy(tflat_h.at[s + j * NUM_ENT],
                                    out_v.at[j], sem)
                   for j in range(NB)]
            for c in cps:
                c.wait()
            ocs = [pltpu.async_copy(out_v.at[j],
                                    nbr_out.at[j, pl.ds(base, _G)], sem2)
                   for j in range(NB)]
            for c in ocs:
                c.wait()

    return k(src, t_flat)


def _gather_stage(src, rel, nbrT, ent_embed, rel_embed):
    mesh = plsc.VectorSubcoreMesh(core_axis_name="c", subcore_axis_name="s",
                                  num_cores=2, num_subcores=16)
    f32 = jnp.float32

    @functools.partial(
        pl.kernel,
        out_type=[
            jax.ShapeDtypeStruct((B, D), f32),
            jax.ShapeDtypeStruct((B, D), f32),
            jax.ShapeDtypeStruct((B, D), f32),
            jax.ShapeDtypeStruct((B, D), f32),
            jax.ShapeDtypeStruct((B, D), f32),
            jax.ShapeDtypeStruct((B, D), f32),
            jax.ShapeDtypeStruct((4, B), f32),
        ],
        mesh=mesh,
        scratch_types=[
            pltpu.VMEM((_G,), jnp.int32),
            pltpu.VMEM((_G, D), f32),
            pltpu.VMEM((_G,), jnp.int32),
            pltpu.VMEM((_G,), jnp.int32),
            pltpu.VMEM((_G,), jnp.int32),
            pltpu.VMEM((3, _G, D), f32),
            pltpu.VMEM((_G, D), f32),
            pltpu.VMEM((_G,), f32),
            pltpu.SemaphoreType.DMA,
            pltpu.SemaphoreType.DMA,
        ],
    )
    def k(src_h, rel_h, tT_h, ent_h, relemb_h,
          h_out, r_out, es0_out, es1_out, es2_out, es3_out, cnt4_out,
          id_v, hr_v, ix0, ix1, ix2, g_v, es_v, cnt_v, sem, sem2):
        wid = lax.axis_index("s") * 2 + lax.axis_index("c")
        grp = wid // 4
        role = wid % 4
        base = grp * _G
        ixs = (ix0, ix1, ix2)
        es_outs = (es0_out, es1_out, es2_out, es3_out)
        id_hs = (None, src_h, rel_h, None)
        emb_hs = (None, ent_h, relemb_h, None)
        row_outs = (None, h_out, r_out, None)

        for rr in range(4):
            @pl.when(role == rr)
            def _(rr=rr):
                jset = _JSETS[rr]
                icps = [pltpu.async_copy(tT_h.at[j, pl.ds(base, _G)],
                                         ixs[kk], sem2)
                        for kk, j in enumerate(jset)]
                if id_hs[rr] is not None:
                    icps.append(pltpu.async_copy(
                        id_hs[rr].at[pl.ds(base, _G)], id_v, sem2))
                for c in icps:
                    c.wait()
                cps = [pltpu.async_copy(ent_h.at[ixs[kk]], g_v.at[kk], sem)
                       for kk in range(len(jset))]
                if id_hs[rr] is not None:
                    cps.append(pltpu.async_copy(
                        emb_hs[rr].at[id_v], hr_v, sem))
                cnt = jnp.where(ix0[...] >= THRESH, 1.0, 0.0)
                for kk in range(1, len(jset)):
                    cnt = cnt + jnp.where(ixs[kk][...] >= THRESH, 1.0, 0.0)
                cnt_v[...] = cnt
                for c in cps:
                    c.wait()

                nj = len(jset)

                def acc_body(i, c):
                    for c8 in range(D // 16):
                        sl = pl.ds(c8 * 16, 16)
                        a = g_v[0, i, sl]
                        for kk in range(1, nj):
                            a = a + g_v[kk, i, sl]
                        es_v[i, sl] = a
                    return c
                lax.fori_loop(0, _G, acc_body, 0)

                ocs = [pltpu.async_copy(es_v, es_outs[rr].at[pl.ds(base, _G)],
                                        sem2),
                       pltpu.async_copy(
                           cnt_v, cnt4_out.at[rr, pl.ds(base, _G)], sem2)]
                if id_hs[rr] is not None:
                    ocs.append(pltpu.async_copy(
                        hr_v, row_outs[rr].at[pl.ds(base, _G)], sem2))
                for c in ocs:
                    c.wait()

    return k(src, rel, nbrT, ent_embed, rel_embed)


def _dense_body(h_ref, r_ref, e0_ref, e1_ref, e2_ref, e3_ref,
                cnt4_ref, w_ref, bias_ref, wve_ref,
                f1a_ref, f1b_ref, b1_ref, f2_ref, b2_ref, ent_ref,
                yct_ref, out_s):
    @pl.when(pl.program_id(0) == 0)
    def _():
        e_sum = e0_ref[...] + e1_ref[...] + e2_ref[...] + e3_ref[...]
        cnt_row = jnp.sum(cnt4_ref[...], axis=0, keepdims=True)
        node_v = jax.lax.dot_general(cnt_row, w_ref[...],
                                     (((0,), (0,)), ((), ())),
                                     preferred_element_type=jnp.float32)
        V = (h_ref[...]
             + jnp.dot(e_sum, wve_ref[...], preferred_element_type=jnp.float32)
             + node_v + bias_ref[...])
        z1 = jnp.maximum(
            jnp.dot(V, f1a_ref[...], preferred_element_type=jnp.float32)
            + jnp.dot(r_ref[...], f1b_ref[...], preferred_element_type=jnp.float32)
            + b1_ref[...], 0.0)
        out_s[...] = (jnp.dot(z1, f2_ref[...], preferred_element_type=jnp.float32)
                      + b2_ref[...])

    logits = jax.lax.dot_general(ent_ref[...], out_s[...],
                                 (((1,), (1,)), ((), ())),
                                 preferred_element_type=jnp.float32)
    yct_ref[...] = jax.nn.sigmoid(logits)


def _dense_stage(h_emb, r_emb, es_parts, cnt4, nod_embed, WV, bV,
                 fc1_w, fc1_b, fc2_w, fc2_b, ent_embed):
    WVnT = WV[:, D:].T
    w_row = ((nod_embed[1] - nod_embed[0]) @ WVnT).reshape(1, D)
    bias_row = (NB * (nod_embed[0] @ WVnT + bV)).reshape(1, D)
    n_chunks = pl.cdiv(NUM_ENT, VOCAB_CHUNK)
    const = lambda shape: pl.BlockSpec(shape, lambda i: (0, 0))
    return pl.pallas_call(
        _dense_body,
        grid=(n_chunks,),
        in_specs=[
            const((B, D)),
            const((B, D)),
            const((B, D)), const((B, D)), const((B, D)), const((B, D)),
            const((4, B)),
            const((1, D)),
            const((1, D)),
            const((D, D)),
            const((D, D)),
            const((D, D)),
            const((1, D)),
            const((D, D)),
            const((1, D)),
            pl.BlockSpec((VOCAB_CHUNK, D), lambda i: (i, 0)),
        ],
        out_specs=pl.BlockSpec((VOCAB_CHUNK, B), lambda i: (i, 0)),
        out_shape=jax.ShapeDtypeStruct((NUM_ENT, B), jnp.float32),
        scratch_shapes=[pltpu.VMEM((B, D), jnp.float32)],
    )(h_emb, r_emb, *es_parts, cnt4, w_row, bias_row,
      WV[:, :D].T,
      fc1_w[:, :D].T, fc1_w[:, D:].T, fc1_b.reshape(1, D),
      fc2_w.T, fc2_b.reshape(1, D), ent_embed)


def kernel(src, rel, t_idxs, ent_embed, rel_embed, nod_embed,
           WQ, bQ, WK, bK, WV, bV, fc1_w, fc1_b, fc2_w, fc2_b):
    nbrT = _nbr_fetch(src, t_idxs.T.reshape(-1))
    outs = _gather_stage(src, rel, nbrT, ent_embed, rel_embed)
    h_emb, r_emb = outs[0], outs[1]
    es_parts, cnt4 = outs[2:6], outs[6]
    yct = _dense_stage(h_emb, r_emb, es_parts, cnt4, nod_embed,
                       WV, bV, fc1_w, fc1_b, fc2_w, fc2_b, ent_embed)
    return yct.T


# speedup vs baseline: 1.4784x; 1.0545x over previous
import functools

import jax
import jax.numpy as jnp
from jax import lax
from jax.experimental import pallas as pl
from jax.experimental.pallas import tpu as pltpu
from jax.experimental.pallas import tpu_sc as plsc

NUM_ENT = 50000
NUM_REL = 474
D = 128
NODE_D = 32
B = 128
NB = 10
THRESH = 1373

VOCAB_CHUNK = 8192

_G = 16
_NG = B // _G
_JSETS = ((0, 1, 2), (3, 4), (5, 6), (7, 8, 9))


def _nbr_fetch(src, t_flat):
    mesh = plsc.VectorSubcoreMesh(core_axis_name="c", subcore_axis_name="s",
                                  num_cores=2, num_subcores=16)

    @functools.partial(
        pl.kernel,
        out_type=jax.ShapeDtypeStruct((NB, B), jnp.int32),
        mesh=mesh,
        compiler_params=pltpu.CompilerParams(use_tc_tiling_on_sc=False),
        scratch_types=[
            pltpu.VMEM((_G,), jnp.int32),
            pltpu.VMEM((NB, _G), jnp.int32),
            pltpu.SemaphoreType.DMA,
            pltpu.SemaphoreType.DMA,
        ],
    )
    def k(src_h, tflat_h, nbr_out, src_v, out_v, sem, sem2):
        wid = lax.axis_index("s") * 2 + lax.axis_index("c")

        @pl.when(wid < _NG)
        def _():
            base = wid * _G
            pltpu.sync_copy(src_h.at[pl.ds(base, _G)], src_v)
            s = src_v[...]
            cps = [pltpu.async_copy(tflat_h.at[s + j * NUM_ENT],
                                    out_v.at[j], sem)
                   for j in range(NB)]
            for c in cps:
                c.wait()
            ocs = [pltpu.async_copy(out_v.at[j],
                                    nbr_out.at[j, pl.ds(base, _G)], sem2)
                   for j in range(NB)]
            for c in ocs:
                c.wait()

    return k(src, t_flat)


def _gather_stage(src, rel, nbrT, ent_embed, rel_embed):
    mesh = plsc.VectorSubcoreMesh(core_axis_name="c", subcore_axis_name="s",
                                  num_cores=2, num_subcores=16)
    f32 = jnp.float32

    @functools.partial(
        pl.kernel,
        out_type=[
            jax.ShapeDtypeStruct((B, D), f32),
            jax.ShapeDtypeStruct((B, D), f32),
            jax.ShapeDtypeStruct((B, D), f32),
            jax.ShapeDtypeStruct((B, D), f32),
            jax.ShapeDtypeStruct((B, D), f32),
            jax.ShapeDtypeStruct((B, D), f32),
            jax.ShapeDtypeStruct((4, B), f32),
        ],
        mesh=mesh,
        scratch_types=[
            pltpu.VMEM((_G,), jnp.int32),
            pltpu.VMEM((_G, D), f32),
            pltpu.VMEM((_G,), jnp.int32),
            pltpu.VMEM((_G,), jnp.int32),
            pltpu.VMEM((_G,), jnp.int32),
            pltpu.VMEM((3, _G, D), f32),
            pltpu.VMEM((_G, D), f32),
            pltpu.VMEM((_G,), f32),
            pltpu.SemaphoreType.DMA,
            pltpu.SemaphoreType.DMA,
        ],
    )
    def k(src_h, rel_h, tT_h, ent_h, relemb_h,
          h_out, r_out, es0_out, es1_out, es2_out, es3_out, cnt4_out,
          id_v, hr_v, ix0, ix1, ix2, g_v, es_v, cnt_v, sem, sem2):
        wid = lax.axis_index("s") * 2 + lax.axis_index("c")
        grp = wid // 4
        role = wid % 4
        base = grp * _G
        ixs = (ix0, ix1, ix2)
        es_outs = (es0_out, es1_out, es2_out, es3_out)
        id_hs = (None, src_h, rel_h, None)
        emb_hs = (None, ent_h, relemb_h, None)
        row_outs = (None, h_out, r_out, None)

        for rr in range(4):
            @pl.when(role == rr)
            def _(rr=rr):
                jset = _JSETS[rr]
                icps = [pltpu.async_copy(tT_h.at[j, pl.ds(base, _G)],
                                         ixs[kk], sem2)
                        for kk, j in enumerate(jset)]
                if id_hs[rr] is not None:
                    icps.append(pltpu.async_copy(
                        id_hs[rr].at[pl.ds(base, _G)], id_v, sem2))
                for c in icps:
                    c.wait()
                cps = [pltpu.async_copy(ent_h.at[ixs[kk]], g_v.at[kk], sem)
                       for kk in range(len(jset))]
                if id_hs[rr] is not None:
                    cps.append(pltpu.async_copy(
                        emb_hs[rr].at[id_v], hr_v, sem))
                cnt = jnp.where(ix0[...] >= THRESH, 1.0, 0.0)
                for kk in range(1, len(jset)):
                    cnt = cnt + jnp.where(ixs[kk][...] >= THRESH, 1.0, 0.0)
                cnt_v[...] = cnt
                for c in cps:
                    c.wait()

                nj = len(jset)

                def acc_body(i, c):
                    for c8 in range(D // 16):
                        sl = pl.ds(c8 * 16, 16)
                        a = g_v[0, i, sl]
                        for kk in range(1, nj):
                            a = a + g_v[kk, i, sl]
                        es_v[i, sl] = a
                    return c
                lax.fori_loop(0, _G, acc_body, 0)

                ocs = [pltpu.async_copy(es_v, es_outs[rr].at[pl.ds(base, _G)],
                                        sem2),
                       pltpu.async_copy(
                           cnt_v, cnt4_out.at[rr, pl.ds(base, _G)], sem2)]
                if id_hs[rr] is not None:
                    ocs.append(pltpu.async_copy(
                        hr_v, row_outs[rr].at[pl.ds(base, _G)], sem2))
                for c in ocs:
                    c.wait()

    return k(src, rel, nbrT, ent_embed, rel_embed)


def _dense_body(h_ref, r_ref, e0_ref, e1_ref, e2_ref, e3_ref,
                cnt4_ref, w_ref, bias_ref, wve_ref,
                f1a_ref, f1b_ref, b1_ref, f2_ref, b2_ref, ent_ref,
                yct_ref, out_s):
    @pl.when(pl.program_id(0) == 0)
    def _():
        e_sum = e0_ref[...] + e1_ref[...] + e2_ref[...] + e3_ref[...]
        cnt_row = jnp.sum(cnt4_ref[...], axis=0, keepdims=True)
        node_v = jax.lax.dot_general(cnt_row, w_ref[...],
                                     (((0,), (0,)), ((), ())),
                                     preferred_element_type=jnp.float32)
        V = (h_ref[...]
             + jnp.dot(e_sum, wve_ref[...], preferred_element_type=jnp.float32)
             + node_v + bias_ref[...])
        z1 = jnp.maximum(
            jnp.dot(V, f1a_ref[...], preferred_element_type=jnp.float32)
            + jnp.dot(r_ref[...], f1b_ref[...], preferred_element_type=jnp.float32)
            + b1_ref[...], 0.0)
        out_s[...] = (jnp.dot(z1, f2_ref[...], preferred_element_type=jnp.float32)
                      + b2_ref[...])

    logits = jax.lax.dot_general(ent_ref[...], out_s[...],
                                 (((1,), (1,)), ((), ())),
                                 preferred_element_type=jnp.float32)
    yct_ref[...] = jax.nn.sigmoid(logits)


def _dense_stage(h_emb, r_emb, es_parts, cnt4, nod_embed, WV, bV,
                 fc1_w, fc1_b, fc2_w, fc2_b, ent_embed):
    WVnT = WV[:, D:].T
    w_row = ((nod_embed[1] - nod_embed[0]) @ WVnT).reshape(1, D)
    bias_row = (NB * (nod_embed[0] @ WVnT + bV)).reshape(1, D)
    n_chunks = pl.cdiv(NUM_ENT, VOCAB_CHUNK)
    const = lambda shape: pl.BlockSpec(shape, lambda i: (0, 0))
    return pl.pallas_call(
        _dense_body,
        grid=(n_chunks,),
        in_specs=[
            const((B, D)),
            const((B, D)),
            const((B, D)), const((B, D)), const((B, D)), const((B, D)),
            const((4, B)),
            const((1, D)),
            const((1, D)),
            const((D, D)),
            const((D, D)),
            const((D, D)),
            const((1, D)),
            const((D, D)),
            const((1, D)),
            pl.BlockSpec((VOCAB_CHUNK, D), lambda i: (i, 0)),
        ],
        out_specs=pl.BlockSpec((VOCAB_CHUNK, B), lambda i: (i, 0)),
        out_shape=jax.ShapeDtypeStruct((NUM_ENT, B), jnp.float32),
        scratch_shapes=[pltpu.VMEM((B, D), jnp.float32)],
    )(h_emb, r_emb, *es_parts, cnt4, w_row, bias_row,
      WV[:, :D].T,
      fc1_w[:, :D].T, fc1_w[:, D:].T, fc1_b.reshape(1, D),
      fc2_w.T, fc2_b.reshape(1, D), ent_embed)


def kernel(src, rel, t_idxs, ent_embed, rel_embed, nod_embed,
           WQ, bQ, WK, bK, WV, bV, fc1_w, fc1_b, fc2_w, fc2_b):
    nbrT = _nbr_fetch(src, t_idxs.T.reshape(-1))
    outs = _gather_stage(src, rel, nbrT, ent_embed, rel_embed)
    h_emb, r_emb = outs[0], outs[1]
    es_parts, cnt4 = outs[2:6], outs[6]
    yct = _dense_stage(h_emb, r_emb, es_parts, cnt4, nod_embed,
                       WV, bV, fc1_w, fc1_b, fc2_w, fc2_b, ent_embed)
    return yct.T
